# baseline (speedup 1.0000x reference)
import jax
import jax.numpy as jnp
from jax.experimental import pallas as pl
from jax.experimental.pallas import tpu as pltpu

_CHUNKS = 2


def _make_copy_kernel(batch):
    def _copy_kernel(tbl, out, buf, in_sem, out_sem):
        num_rows = buf.shape[0]
        blk = num_rows // _CHUNKS

        def in_copy(c):
            sl = pl.ds(c * blk, blk)
            return pltpu.make_async_copy(
                tbl.at[sl, :], buf.at[sl, :], in_sem.at[c])

        def out_copy(b, c):
            sl = pl.ds(c * blk, blk)
            return pltpu.make_async_copy(
                buf.at[sl, :], out.at[b, sl, :], out_sem.at[b, c])

        for c in range(_CHUNKS):
            in_copy(c).start()
        for c in range(_CHUNKS):
            in_copy(c).wait()
            for b in range(batch):
                out_copy(b, c).start()
        for c in range(_CHUNKS):
            for b in range(batch):
                out_copy(b, c).wait()

    return _copy_kernel


def _bcast(table, batch):
    num_rows, dim = table.shape
    return pl.pallas_call(
        _make_copy_kernel(batch),
        in_specs=[pl.BlockSpec(memory_space=pl.ANY)],
        out_specs=pl.BlockSpec(memory_space=pl.ANY),
        out_shape=jax.ShapeDtypeStruct((batch, num_rows, dim), table.dtype),
        scratch_shapes=[
            pltpu.VMEM((num_rows, dim), table.dtype),
            pltpu.SemaphoreType.DMA((_CHUNKS,)),
            pltpu.SemaphoreType.DMA((batch, _CHUNKS)),
        ],
    )(table)


def kernel(inputs, table):
    del inputs
    a = _bcast(table, 2)
    b = _bcast(table, 2)
    return jnp.concatenate([a, b], axis=0)


# speedup vs baseline: 1.4745x; 1.4745x over previous
import functools
import jax
import jax.numpy as jnp
from jax import lax
from jax.experimental import pallas as pl
from jax.experimental.pallas import tpu as pltpu
from jax.experimental.pallas import tpu_sc as plsc

_BATCH = 4
_CH = 32


def kernel(inputs, table):
    del inputs
    num_rows, dim = table.shape
    info = plsc.get_sparse_core_info()
    nw = info.num_cores * info.num_subcores
    rows_per_w = num_rows // nw
    n_chunks = rows_per_w // _CH
    mesh = plsc.VectorSubcoreMesh(core_axis_name="c", subcore_axis_name="s")

    @functools.partial(
        pl.kernel, mesh=mesh,
        out_type=jax.ShapeDtypeStruct((_BATCH, num_rows, dim), table.dtype),
        scratch_types=[
            pltpu.VMEM((_CH, dim), table.dtype),
            pltpu.SemaphoreType.DMA,
        ],
    )
    def _sc_copy(table_hbm, out_hbm, buf, sem):
        wid = lax.axis_index("s") * info.num_cores + lax.axis_index("c")
        base = wid * rows_per_w

        def body(i, carry):
            row0 = base + i * _CH
            pltpu.sync_copy(table_hbm.at[pl.ds(row0, _CH), :], buf)
            for b in range(_BATCH):
                pltpu.sync_copy(buf, out_hbm.at[b, pl.ds(row0, _CH), :])
            return carry

        lax.fori_loop(0, n_chunks, body, 0)

    return _sc_copy(table)


# speedup vs baseline: 2.3051x; 1.5633x over previous
import jax
import jax.numpy as jnp
from jax.experimental import pallas as pl
from jax.experimental.pallas import tpu as pltpu

_BATCH = 4
_CHUNKS = 2


def _copy_kernel(tbl, out, buf, in_sem, out_sem):
    num_rows = buf.shape[0]
    blk = num_rows // _CHUNKS

    def in_copy(c):
        sl = pl.ds(c * blk, blk)
        return pltpu.make_async_copy(tbl.at[sl, :], buf.at[sl, :], in_sem.at[c])

    def out_copy(b, c):
        sl = pl.ds(c * blk, blk)
        return pltpu.make_async_copy(
            buf.at[sl, :], out.at[b, sl, :], out_sem.at[b, c])

    for c in range(_CHUNKS):
        in_copy(c).start()
    for c in range(_CHUNKS):
        in_copy(c).wait()
        for b in range(_BATCH):
            out_copy(b, c).start()
    for c in range(_CHUNKS):
        for b in range(_BATCH):
            out_copy(b, c).wait()


def kernel(inputs, table):
    del inputs
    num_rows, dim = table.shape
    out = pl.pallas_call(
        _copy_kernel,
        in_specs=[pl.BlockSpec(memory_space=pl.ANY)],
        out_specs=pl.BlockSpec(memory_space=pl.ANY),
        out_shape=jax.ShapeDtypeStruct((_BATCH, num_rows, dim), table.dtype),
        scratch_shapes=[
            pltpu.VMEM((num_rows, dim), table.dtype),
            pltpu.SemaphoreType.DMA((_CHUNKS,)),
            pltpu.SemaphoreType.DMA((_BATCH, _CHUNKS)),
        ],
    )(table)
    return out
